# baseline (speedup 1.0000x reference)
import functools

import jax
import jax.numpy as jnp
from jax import lax
from jax.experimental import pallas as pl
from jax.experimental.pallas import tpu as pltpu
from jax.experimental.pallas import tpu_sc as plsc

_B = 16384
_D = 64
_V = 100000
_H1 = 256
_CPT = 4
_HALF = _B // 2

_LO_LEN = 50048
_HI_OFF = 49920
_HI_LEN = 50048
_TAIL0 = _V - 128
_NT = 128

_CBLK = 4096


def _sc_gather_t(uidx, midx, uemb_t, memb_t, tail_u, tail_m):
    mesh = plsc.VectorSubcoreMesh(core_axis_name="c", subcore_axis_name="s")

    @functools.partial(
        pl.kernel,
        mesh=mesh,
        out_type=[
            jax.ShapeDtypeStruct((_D, _B), jnp.float32),
            jax.ShapeDtypeStruct((_D, _B), jnp.float32),
        ],
        scratch_types=[
            pltpu.VMEM((_LO_LEN,), jnp.float32),
            pltpu.VMEM((_HI_LEN,), jnp.float32),
            pltpu.VMEM((_B,), jnp.int32),
            pltpu.VMEM((_HALF,), jnp.float32),
            pltpu.VMEM((_CPT * _NT,), jnp.float32),
            pltpu.SemaphoreType.DMA,
            pltpu.SemaphoreType.DMA,
        ],
        compiler_params=pltpu.CompilerParams(
            use_tc_tiling_on_sc=True, needs_layout_passes=False),
    )
    def k(uidx_hbm, midx_hbm, uemb_hbm, memb_hbm, tailu_hbm, tailm_hbm,
          out_u, out_m, a_v, b_v, idx_v, row_v, tail_v, sem_a, sem_b):
        wid = lax.axis_index("s") * 2 + lax.axis_index("c")
        slot = wid // 2
        is_user = (wid % 2) == 0

        def pass_a(o):
            @plsc.parallel_loop(0, _HALF, 16, unroll=8)
            def _(i):
                iv = idx_v[pl.ds(o * _HALF + i, 16)]
                row_v[pl.ds(i, 16)] = plsc.load_gather(
                    a_v, [jnp.minimum(iv, _LO_LEN - 1)])

        def pass_bt(o, jj):
            @plsc.parallel_loop(0, _HALF, 16, unroll=8)
            def _(i):
                iv = idx_v[pl.ds(o * _HALF + i, 16)]
                gb = plsc.load_gather(
                    b_v, [jnp.clip(iv - _HI_OFF, 0, _HI_LEN - 1)])
                gt = plsc.load_gather(
                    tail_v, [jnp.maximum(iv - _TAIL0, 0) + jj * _NT])
                prev = row_v[pl.ds(i, 16)]
                r = jnp.where(iv >= _LO_LEN, gb, prev)
                row_v[pl.ds(i, 16)] = jnp.where(iv >= _TAIL0, gt, r)

        def work(idx_hbm, tab_hbm, tail_hbm, out_hbm):
            pltpu.sync_copy(idx_hbm, idx_v)
            c0 = slot * _CPT
            for jj in range(_CPT):
                pltpu.sync_copy(tail_hbm.at[c0 + jj],
                                tail_v.at[pl.ds(jj * _NT, _NT)])
            pltpu.async_copy(
                tab_hbm.at[c0].at[pl.ds(0, _LO_LEN)], a_v, sem_a)
            for j in range(_CPT):
                c = c0 + j
                pltpu.make_async_copy(
                    tab_hbm.at[c].at[pl.ds(0, _LO_LEN)], a_v, sem_a).wait()
                pltpu.async_copy(
                    tab_hbm.at[c].at[pl.ds(_HI_OFF, _HI_LEN)], b_v, sem_b)
                pass_a(0)
                pltpu.make_async_copy(
                    tab_hbm.at[c].at[pl.ds(_HI_OFF, _HI_LEN)],
                    b_v, sem_b).wait()
                pass_bt(0, j)
                pltpu.sync_copy(row_v, out_hbm.at[c, pl.ds(0, _HALF)])
                pass_a(1)
                if j + 1 < _CPT:
                    pltpu.async_copy(
                        tab_hbm.at[c + 1].at[pl.ds(0, _LO_LEN)], a_v, sem_a)
                pass_bt(1, j)
                pltpu.sync_copy(row_v, out_hbm.at[c, pl.ds(_HALF, _HALF)])

        @pl.when(is_user)
        def _():
            work(uidx_hbm, uemb_hbm, tailu_hbm, out_u)

        @pl.when(jnp.logical_not(is_user))
        def _():
            work(midx_hbm, memb_hbm, tailm_hbm, out_m)

    return k(uidx, midx, uemb_t, memb_t, tail_u, tail_m)


def _mlp_body(u_ref, m_ref, w1_ref, b1_ref, w2_ref, b2_ref, out_ref):
    w1u = w1_ref[:, :_D]
    w1m = w1_ref[:, _D:]
    h = (jnp.dot(w1u, u_ref[...], preferred_element_type=jnp.float32)
         + jnp.dot(w1m, m_ref[...], preferred_element_type=jnp.float32)
         + b1_ref[...])
    h = jnp.maximum(h, 0.0)
    res = jnp.dot(w2_ref[...], h, preferred_element_type=jnp.float32)
    out_ref[...] = res[0, :] + b2_ref[0, 0]


def _tc_mlp(u_t, m_t, W1, b1_2d, W2, b2_2d):
    grid = (_B // _CBLK,)
    return pl.pallas_call(
        _mlp_body,
        grid=grid,
        in_specs=[
            pl.BlockSpec((_D, _CBLK), lambda i: (0, i)),
            pl.BlockSpec((_D, _CBLK), lambda i: (0, i)),
            pl.BlockSpec((_H1, 2 * _D), lambda i: (0, 0)),
            pl.BlockSpec((_H1, 1), lambda i: (0, 0)),
            pl.BlockSpec((1, _H1), lambda i: (0, 0)),
            pl.BlockSpec((1, 1), lambda i: (0, 0)),
        ],
        out_specs=pl.BlockSpec((_CBLK,), lambda i: (i,)),
        out_shape=jax.ShapeDtypeStruct((_B,), jnp.float32),
        compiler_params=pltpu.CompilerParams(
            dimension_semantics=("parallel",)),
    )(u_t, m_t, W1, b1_2d, W2, b2_2d)


def kernel(x, user_emb, movie_emb, W1, b1, W2, b2):
    uidx = x[0].astype(jnp.int32)
    midx = x[1].astype(jnp.int32)
    uemb_t = user_emb.T
    memb_t = movie_emb.T
    u_t, m_t = _sc_gather_t(uidx, midx, uemb_t, memb_t,
                            uemb_t[:, _TAIL0:], memb_t[:, _TAIL0:])
    return _tc_mlp(u_t, m_t, W1, b1.reshape(_H1, 1), W2, b2.reshape(1, 1))


# speedup vs baseline: 1.2144x; 1.2144x over previous
import functools

import jax
import jax.numpy as jnp
from jax import lax
from jax.experimental import pallas as pl
from jax.experimental.pallas import tpu as pltpu
from jax.experimental.pallas import tpu_sc as plsc

_B = 16384
_D = 64
_V = 100000
_H1 = 256
_COLS_PER_TILE = _D // 16
_HALF = _B // 2

_CBLK = 4096


def _sc_gather_t(uidx, midx, uemb_t, memb_t):
    mesh = plsc.VectorSubcoreMesh(core_axis_name="c", subcore_axis_name="s")

    @functools.partial(
        pl.kernel,
        mesh=mesh,
        out_type=[
            jax.ShapeDtypeStruct((_D, _B), jnp.float32),
            jax.ShapeDtypeStruct((_D, _B), jnp.float32),
        ],
        scratch_types=[
            pltpu.VMEM((_V,), jnp.float32),
            pltpu.VMEM((_B,), jnp.int32),
            pltpu.VMEM((_HALF,), jnp.float32),
        ],
        compiler_params=pltpu.CompilerParams(
            use_tc_tiling_on_sc=True, needs_layout_passes=False),
    )
    def k(uidx_hbm, midx_hbm, uemb_hbm, memb_hbm, out_u, out_m,
          col_v, idx_v, row_v):
        wid = lax.axis_index("s") * 2 + lax.axis_index("c")
        slot = wid // 2
        is_user = (wid % 2) == 0


        def work(idx_hbm, tab_hbm, out_hbm):
            pltpu.sync_copy(idx_hbm, idx_v)
            for j in range(_COLS_PER_TILE):
                c = slot * _COLS_PER_TILE + j
                pltpu.sync_copy(tab_hbm.at[c], col_v)
                for h in range(2):
                    @plsc.parallel_loop(0, _HALF, 16, unroll=16)
                    def _(i):
                        iv = idx_v[pl.ds(h * _HALF + i, 16)]
                        row_v[pl.ds(i, 16)] = plsc.load_gather(
                            col_v, [iv])
                    pltpu.sync_copy(
                        row_v, out_hbm.at[c, pl.ds(h * _HALF, _HALF)])

        @pl.when(is_user)
        def _():
            work(uidx_hbm, uemb_hbm, out_u)

        @pl.when(jnp.logical_not(is_user))
        def _():
            work(midx_hbm, memb_hbm, out_m)

    return k(uidx, midx, uemb_t, memb_t)


def _mlp_body(u_ref, m_ref, w1_ref, b1_ref, w2_ref, b2_ref, out_ref):
    w1u = w1_ref[:, :_D]
    w1m = w1_ref[:, _D:]
    h = (jnp.dot(w1u, u_ref[...], preferred_element_type=jnp.float32)
         + jnp.dot(w1m, m_ref[...], preferred_element_type=jnp.float32)
         + b1_ref[...])
    h = jnp.maximum(h, 0.0)
    res = jnp.dot(w2_ref[...], h, preferred_element_type=jnp.float32)
    out_ref[...] = res[0, :] + b2_ref[0, 0]


def _tc_mlp(u_t, m_t, W1, b1_2d, W2, b2_2d):
    grid = (_B // _CBLK,)
    return pl.pallas_call(
        _mlp_body,
        grid=grid,
        in_specs=[
            pl.BlockSpec((_D, _CBLK), lambda i: (0, i)),
            pl.BlockSpec((_D, _CBLK), lambda i: (0, i)),
            pl.BlockSpec((_H1, 2 * _D), lambda i: (0, 0)),
            pl.BlockSpec((_H1, 1), lambda i: (0, 0)),
            pl.BlockSpec((1, _H1), lambda i: (0, 0)),
            pl.BlockSpec((1, 1), lambda i: (0, 0)),
        ],
        out_specs=pl.BlockSpec((_CBLK,), lambda i: (i,)),
        out_shape=jax.ShapeDtypeStruct((_B,), jnp.float32),
        compiler_params=pltpu.CompilerParams(
            dimension_semantics=("parallel",)),
    )(u_t, m_t, W1, b1_2d, W2, b2_2d)


def kernel(x, user_emb, movie_emb, W1, b1, W2, b2):
    uidx = x[0].astype(jnp.int32)
    midx = x[1].astype(jnp.int32)
    u_t, m_t = _sc_gather_t(uidx, midx, user_emb.T, movie_emb.T)
    return _tc_mlp(u_t, m_t, W1, b1.reshape(_H1, 1), W2, b2.reshape(1, 1))
